# baseline (speedup 1.0000x reference)
import jax
import jax.numpy as jnp
from jax.experimental import pallas as pl
from jax.experimental.pallas import tpu as pltpu

TM = 200


def _fused_kernel(x_ref, adja_ref, adjb_ref, w1_ref, b1_ref, w2_ref, b2_ref,
                  wmu_ref, bmu_ref, wlv_ref, blv_ref,
                  mu_ref, lv_ref, s_ref, h_ref):
    p = pl.program_id(0)
    i = pl.program_id(1)

    @pl.when(jnp.logical_and(p == 0, i == 0))
    def _seed():
        s_ref[0] = jax.lax.dot_general(
            x_ref[...], w1_ref[...], (((1,), (0,)), ((), ())),
            preferred_element_type=jnp.float32)

    s = s_ref[p]

    @pl.when(i % 2 == 0)
    def _even():
        h_ref[...] = jax.lax.dot_general(
            adja_ref[...], s, (((1,), (0,)), ((), ())),
            preferred_element_type=jnp.float32)

    @pl.when(i % 2 == 1)
    def _odd():
        h_ref[...] = jax.lax.dot_general(
            adjb_ref[...], s, (((1,), (0,)), ((), ())),
            preferred_element_type=jnp.float32)

    b = jnp.where(p == 0, b1_ref[...], b2_ref[...])
    h = jnp.maximum(h_ref[...] + b, 0.0)

    @pl.when(p == 0)
    def _phase0():
        s_ref[1, pl.ds(i * TM, TM), :] = jax.lax.dot_general(
            h, w2_ref[...], (((1,), (0,)), ((), ())),
            preferred_element_type=jnp.float32)

    @pl.when(p == 1)
    def _phase1():
        mu_ref[...] = jax.lax.dot_general(
            h, wmu_ref[...], (((1,), (0,)), ((), ())),
            preferred_element_type=jnp.float32) + bmu_ref[...]
        lv_ref[...] = jax.lax.dot_general(
            h, wlv_ref[...], (((1,), (0,)), ((), ())),
            preferred_element_type=jnp.float32) + blv_ref[...]


def kernel(x, adj, W1, b1, W2, b2, Wmu, bmu, Wlv, blv):
    n, nfeat = x.shape
    nhid = W1.shape[1]
    latent = Wmu.shape[1]
    nb = n // TM

    full = lambda p, i: (0, 0)
    adj_a = lambda p, i: (jnp.where(i % 2 == 0, i, jnp.where(i == nb - 1, 0, i + 1)), 0)
    adj_b = lambda p, i: (jnp.where(i % 2 == 1, i, jnp.where(i == nb - 1, 0, i + 1)), 0)
    out_tile = lambda p, i: (jnp.where(p == 0, 0, i), 0)

    mu, lv = pl.pallas_call(
        _fused_kernel,
        grid=(2, nb),
        in_specs=[
            pl.BlockSpec((n, nfeat), full),
            pl.BlockSpec((TM, n), adj_a),
            pl.BlockSpec((TM, n), adj_b),
            pl.BlockSpec((nfeat, nhid), full),
            pl.BlockSpec((1, nhid), full),
            pl.BlockSpec((nhid, nhid), full),
            pl.BlockSpec((1, nhid), full),
            pl.BlockSpec((nhid, latent), full),
            pl.BlockSpec((1, latent), full),
            pl.BlockSpec((nhid, latent), full),
            pl.BlockSpec((1, latent), full),
        ],
        out_specs=[
            pl.BlockSpec((TM, latent), out_tile),
            pl.BlockSpec((TM, latent), out_tile),
        ],
        out_shape=[
            jax.ShapeDtypeStruct((n, latent), jnp.float32),
            jax.ShapeDtypeStruct((n, latent), jnp.float32),
        ],
        scratch_shapes=[
            pltpu.VMEM((2, n, nhid), jnp.float32),
            pltpu.VMEM((TM, nhid), jnp.float32),
        ],
    )(x, adj, adj, W1, b1.reshape(1, nhid), W2, b2.reshape(1, nhid),
      Wmu, bmu.reshape(1, latent), Wlv, blv.reshape(1, latent))

    return (mu, lv)


# speedup vs baseline: 1.1172x; 1.1172x over previous
import jax
import jax.numpy as jnp
from jax.experimental import pallas as pl
from jax.experimental.pallas import tpu as pltpu

N = 10000
TM = 200
NB = N // TM
NBUF = 4
TOTAL = 2 * NB


def _fetch(adj_hbm, ring, sems, g):
    slot = jax.lax.rem(g, NBUF)
    row = jax.lax.rem(g, NB) * TM
    pltpu.make_async_copy(
        adj_hbm.at[pl.ds(row, TM), :], ring.at[slot], sems.at[slot]).start()


def _fused_kernel(x_ref, adj_hbm, w1_ref, b1_ref, w2_ref, b2_ref,
                  wmu_ref, bmu_ref, wlv_ref, blv_ref,
                  mu_ref, lv_ref, s_ref, ring, sems):
    p = pl.program_id(0)
    i = pl.program_id(1)
    g = p * NB + i

    @pl.when(g == 0)
    def _prologue():
        for k in range(NBUF - 1):
            _fetch(adj_hbm, ring, sems, k)
        s_ref[0] = jax.lax.dot_general(
            x_ref[...], w1_ref[...], (((1,), (0,)), ((), ())),
            preferred_element_type=jnp.float32)

    @pl.when(g + NBUF - 1 < TOTAL)
    def _prefetch():
        _fetch(adj_hbm, ring, sems, g + NBUF - 1)

    slot = jax.lax.rem(g, NBUF)
    pltpu.make_async_copy(
        adj_hbm.at[pl.ds(jax.lax.rem(g, NB) * TM, TM), :],
        ring.at[slot], sems.at[slot]).wait()

    s = s_ref[p]
    h = jax.lax.dot_general(
        ring[slot], s, (((1,), (0,)), ((), ())),
        preferred_element_type=jnp.float32)
    b = jnp.where(p == 0, b1_ref[...], b2_ref[...])
    h = jnp.maximum(h + b, 0.0)

    @pl.when(p == 0)
    def _phase0():
        s_ref[1, pl.ds(i * TM, TM), :] = jax.lax.dot_general(
            h, w2_ref[...], (((1,), (0,)), ((), ())),
            preferred_element_type=jnp.float32)

    @pl.when(p == 1)
    def _phase1():
        mu_ref[...] = jax.lax.dot_general(
            h, wmu_ref[...], (((1,), (0,)), ((), ())),
            preferred_element_type=jnp.float32) + bmu_ref[...]
        lv_ref[...] = jax.lax.dot_general(
            h, wlv_ref[...], (((1,), (0,)), ((), ())),
            preferred_element_type=jnp.float32) + blv_ref[...]


def kernel(x, adj, W1, b1, W2, b2, Wmu, bmu, Wlv, blv):
    n, nfeat = x.shape
    nhid = W1.shape[1]
    latent = Wmu.shape[1]

    full = lambda p, i: (0, 0)
    out_tile = lambda p, i: (jnp.where(p == 0, 0, i), 0)

    mu, lv = pl.pallas_call(
        _fused_kernel,
        grid=(2, NB),
        in_specs=[
            pl.BlockSpec((n, nfeat), full),
            pl.BlockSpec(memory_space=pl.ANY),
            pl.BlockSpec((nfeat, nhid), full),
            pl.BlockSpec((1, nhid), full),
            pl.BlockSpec((nhid, nhid), full),
            pl.BlockSpec((1, nhid), full),
            pl.BlockSpec((nhid, latent), full),
            pl.BlockSpec((1, latent), full),
            pl.BlockSpec((nhid, latent), full),
            pl.BlockSpec((1, latent), full),
        ],
        out_specs=[
            pl.BlockSpec((TM, latent), out_tile),
            pl.BlockSpec((TM, latent), out_tile),
        ],
        out_shape=[
            jax.ShapeDtypeStruct((n, latent), jnp.float32),
            jax.ShapeDtypeStruct((n, latent), jnp.float32),
        ],
        scratch_shapes=[
            pltpu.VMEM((2, n, nhid), jnp.float32),
            pltpu.VMEM((NBUF, TM, N), jnp.float32),
            pltpu.SemaphoreType.DMA((NBUF,)),
        ],
    )(x, adj, W1, b1.reshape(1, nhid), W2, b2.reshape(1, nhid),
      Wmu, bmu.reshape(1, latent), Wlv, blv.reshape(1, latent))

    return (mu, lv)


# speedup vs baseline: 1.1177x; 1.0004x over previous
import jax
import jax.numpy as jnp
from jax.experimental import pallas as pl
from jax.experimental.pallas import tpu as pltpu

N = 10000
TM = 200
NB = N // TM
NBUF = 4
TOTAL = 2 * NB
Q = 5
TMQ = TM // Q


def _fetch(adj_hbm, ring, sems, g):
    slot = jax.lax.rem(g, NBUF)
    row = jax.lax.rem(g, NB) * TM
    for q in range(Q):
        pltpu.make_async_copy(
            adj_hbm.at[pl.ds(row + q * TMQ, TMQ), :],
            ring.at[slot, pl.ds(q * TMQ, TMQ), :],
            sems.at[slot, q]).start()


def _wait(adj_hbm, ring, sems, g):
    slot = jax.lax.rem(g, NBUF)
    row = jax.lax.rem(g, NB) * TM
    for q in range(Q):
        pltpu.make_async_copy(
            adj_hbm.at[pl.ds(row + q * TMQ, TMQ), :],
            ring.at[slot, pl.ds(q * TMQ, TMQ), :],
            sems.at[slot, q]).wait()


def _fused_kernel(x_ref, adj_hbm, w1_ref, b1_ref, w2_ref, b2_ref,
                  wmu_ref, bmu_ref, wlv_ref, blv_ref,
                  mu_ref, lv_ref, s_ref, ring, sems):
    p = pl.program_id(0)
    i = pl.program_id(1)
    g = p * NB + i

    @pl.when(g == 0)
    def _prologue():
        for k in range(NBUF - 1):
            _fetch(adj_hbm, ring, sems, k)
        s_ref[0] = jax.lax.dot_general(
            x_ref[...], w1_ref[...], (((1,), (0,)), ((), ())),
            preferred_element_type=jnp.float32)

    @pl.when(g + NBUF - 1 < TOTAL)
    def _prefetch():
        _fetch(adj_hbm, ring, sems, g + NBUF - 1)

    slot = jax.lax.rem(g, NBUF)
    _wait(adj_hbm, ring, sems, g)

    s = s_ref[p]
    h = jax.lax.dot_general(
        ring[slot], s, (((1,), (0,)), ((), ())),
        preferred_element_type=jnp.float32)
    b = jnp.where(p == 0, b1_ref[...], b2_ref[...])
    h = jnp.maximum(h + b, 0.0)

    @pl.when(p == 0)
    def _phase0():
        s_ref[1, pl.ds(i * TM, TM), :] = jax.lax.dot_general(
            h, w2_ref[...], (((1,), (0,)), ((), ())),
            preferred_element_type=jnp.float32)

    @pl.when(p == 1)
    def _phase1():
        mu_ref[...] = jax.lax.dot_general(
            h, wmu_ref[...], (((1,), (0,)), ((), ())),
            preferred_element_type=jnp.float32) + bmu_ref[...]
        lv_ref[...] = jax.lax.dot_general(
            h, wlv_ref[...], (((1,), (0,)), ((), ())),
            preferred_element_type=jnp.float32) + blv_ref[...]


def kernel(x, adj, W1, b1, W2, b2, Wmu, bmu, Wlv, blv):
    n, nfeat = x.shape
    nhid = W1.shape[1]
    latent = Wmu.shape[1]

    full = lambda p, i: (0, 0)
    out_tile = lambda p, i: (jnp.where(p == 0, 0, i), 0)

    mu, lv = pl.pallas_call(
        _fused_kernel,
        grid=(2, NB),
        in_specs=[
            pl.BlockSpec((n, nfeat), full),
            pl.BlockSpec(memory_space=pl.ANY),
            pl.BlockSpec((nfeat, nhid), full),
            pl.BlockSpec((1, nhid), full),
            pl.BlockSpec((nhid, nhid), full),
            pl.BlockSpec((1, nhid), full),
            pl.BlockSpec((nhid, latent), full),
            pl.BlockSpec((1, latent), full),
            pl.BlockSpec((nhid, latent), full),
            pl.BlockSpec((1, latent), full),
        ],
        out_specs=[
            pl.BlockSpec((TM, latent), out_tile),
            pl.BlockSpec((TM, latent), out_tile),
        ],
        out_shape=[
            jax.ShapeDtypeStruct((n, latent), jnp.float32),
            jax.ShapeDtypeStruct((n, latent), jnp.float32),
        ],
        scratch_shapes=[
            pltpu.VMEM((2, n, nhid), jnp.float32),
            pltpu.VMEM((NBUF, TM, N), jnp.float32),
            pltpu.SemaphoreType.DMA((NBUF, Q)),
        ],
    )(x, adj, W1, b1.reshape(1, nhid), W2, b2.reshape(1, nhid),
      Wmu, bmu.reshape(1, latent), Wlv, blv.reshape(1, latent))

    return (mu, lv)
